# baseline (speedup 1.0000x reference)
import functools

import jax
import jax.numpy as jnp
from jax import lax
from jax.experimental import pallas as pl
from jax.experimental.pallas import tpu as pltpu
from jax.experimental.pallas import tpu_sc as plsc

N_NODES = 10000
N_EDGES = 160000
F = 16

NC = 2
NS = 16
NW = NC * NS
GROUP = 128
NGROUPS = 40
E_PER_W = GROUP * NGROUPS
E_REAL_W = N_EDGES // NW
N_ACC = 10240
DUMMY = N_ACC - 1


def _prep_body(x_ref, wnn_ref, bnn_ref, y_ref, c_ref):
    k = lax.broadcasted_iota(jnp.int32, (F * F, F), 0)
    i = lax.broadcasted_iota(jnp.int32, (F * F, F), 1)
    sel = jnp.where(k // F == i, 1.0, 0.0)
    wsum = wnn_ref[...] @ sel
    bsum = bnn_ref[...] @ sel
    xv = x_ref[...]
    y_ref[...] = lax.dot_general(xv, wsum, (((1,), (1,)), ((), ())))
    c_ref[...] = jnp.sum(xv * bsum, axis=1, keepdims=True)


_prep = pl.pallas_call(
    _prep_body,
    out_shape=(
        jax.ShapeDtypeStruct((N_NODES, F), jnp.float32),
        jax.ShapeDtypeStruct((N_NODES, 1), jnp.float32),
    ),
)


def _final_body(x_ref, root_ref, bias_ref, t_ref, c_ref, o_ref):
    tot = t_ref[0] + t_ref[1]
    cnt = c_ref[0] + c_ref[1]
    mean = jnp.where(cnt > 0, tot / (cnt * float(F)), 0.0)
    o_ref[...] = x_ref[...] @ root_ref[...] + bias_ref[...] + mean


_final = pl.pallas_call(
    _final_body,
    out_shape=jax.ShapeDtypeStruct((N_NODES, F), jnp.float32),
    grid=(1,),
    in_specs=[
        pl.BlockSpec((N_NODES, F), lambda i: (0, 0)),
        pl.BlockSpec((F, F), lambda i: (0, 0)),
        pl.BlockSpec((1, F), lambda i: (0, 0)),
        pl.BlockSpec((NC, N_NODES, 1), lambda i: (0, 0, 0)),
        pl.BlockSpec((NC, N_NODES, 1), lambda i: (0, 0, 0)),
    ],
    out_specs=pl.BlockSpec((N_NODES, F), lambda i: (0, 0)),
)


def _sc_body(ei_hbm, ea_hbm, y_hbm, c_hbm,
             tot_out, cnt_out,
             row_v, col1_v, col_v, ea_v, yr_v, s_v, c_v, ones_v, zv,
             tot_sh, cnt_sh, sem_st, sem_g, sem_s):
    cid = lax.axis_index("c")
    sid = lax.axis_index("s")
    wid = cid * NS + sid

    iota = lax.iota(jnp.int32, 16)
    zeros16 = jnp.zeros((16,), jnp.float32)
    ones16 = jnp.ones((16,), jnp.float32)
    zeros16i = jnp.zeros((16,), jnp.int32)
    dummy16 = jnp.full((16,), DUMMY, jnp.int32)

    ntail = (E_PER_W - E_REAL_W) // 16 + 1
    for k in range(ntail):
        row_v[pl.ds(E_PER_W - 16 * ntail + 16 * k, 16)] = zeros16i
        col1_v[pl.ds(E_PER_W - 16 * ntail + 16 * k, 16)] = dummy16

    cp0 = pltpu.async_copy(
        ei_hbm.at[0].at[pl.ds(wid * E_REAL_W, E_REAL_W)],
        row_v.at[pl.ds(0, E_REAL_W)], sem_st)
    cp1 = pltpu.async_copy(
        ei_hbm.at[1].at[pl.ds(wid * E_REAL_W, E_REAL_W)],
        col1_v.at[pl.ds(0, E_REAL_W)], sem_st)
    cp2 = pltpu.async_copy(c_hbm, c_v, sem_st)
    cp3 = pltpu.async_copy(
        ea_hbm.at[pl.ds(wid * E_REAL_W, E_REAL_W)],
        ea_v.at[pl.ds(0, E_REAL_W)], sem_st)

    for k in range(N_ACC // NS // 16):
        zv[pl.ds(k * 16, 16)] = zeros16
    for k in range(GROUP // 16):
        ones_v[pl.ds(k * 16, 16)] = ones16
    cp0.wait()
    cp1.wait()
    cp2.wait()
    cp3.wait()
    for t in range(E_PER_W // 16):
        col_v[t // 8, pl.ds((t % 8) * 16, 16)] = col1_v[pl.ds(t * 16, 16)]
    pltpu.sync_copy(zv, tot_sh.at[pl.ds(sid * (N_ACC // NS), N_ACC // NS)])
    pltpu.sync_copy(zv, cnt_sh.at[pl.ds(sid * (N_ACC // NS), N_ACC // NS)])
    plsc.subcore_barrier()

    def gather_start(g, base):
        pltpu.async_copy(
            y_hbm.at[row_v.at[pl.ds(g * GROUP, GROUP)]],
            yr_v.at[pl.ds(base, GROUP)], sem_g)

    def gather_wait(g, base):
        pltpu.make_async_copy(
            y_hbm.at[row_v.at[pl.ds(g * GROUP, GROUP)]],
            yr_v.at[pl.ds(base, GROUP)], sem_g).wait()

    def compute_group(g, base):
        for j in range(GROUP // 16):
            el = iota + (j * 16)
            eg = el + g * GROUP
            ey = el + base
            ridx = plsc.load_gather(row_v, [eg])
            acc = plsc.load_gather(c_v, [ridx])
            for i in range(F):
                ii = jnp.full((16,), i, jnp.int32)
                yc = plsc.load_gather(yr_v, [ey, ii])
                ec = plsc.load_gather(ea_v, [eg, ii])
                acc = acc + yc * ec
            s_v[g, pl.ds(j * 16, 16)] = acc
        pltpu.async_copy(s_v.at[g], tot_sh.at[col_v.at[g]], sem_s, add=True)
        pltpu.async_copy(ones_v, cnt_sh.at[col_v.at[g]], sem_s, add=True)

    gather_start(0, 0)

    def pair_body(k, _):
        g0 = k * 2
        g1 = g0 + 1
        gather_wait(g0, 0)
        gather_start(g1, GROUP)
        compute_group(g0, 0)
        gather_wait(g1, GROUP)
        gather_start(jnp.minimum(g0 + 2, NGROUPS - 2), 0)
        compute_group(g1, GROUP)
        return ()

    lax.fori_loop(0, NGROUPS // 2, pair_body, ())
    gather_wait(NGROUPS - 2, 0)

    def drain_body(g, _):
        pltpu.make_async_copy(s_v.at[g], tot_sh.at[col_v.at[g]], sem_s).wait()
        pltpu.make_async_copy(ones_v, cnt_sh.at[col_v.at[g]], sem_s).wait()
        return ()

    lax.fori_loop(0, NGROUPS, drain_body, ())
    plsc.subcore_barrier()

    @pl.when(sid == 0)
    def _():
        pltpu.sync_copy(tot_sh, tot_out.at[cid])
        pltpu.sync_copy(cnt_sh, cnt_out.at[cid])


@functools.cache
def _get_sc_core():
  return functools.partial(
    pl.kernel,
    out_type=(
        jax.ShapeDtypeStruct((NC, N_ACC), jnp.float32),
        jax.ShapeDtypeStruct((NC, N_ACC), jnp.float32),
    ),
    mesh=plsc.VectorSubcoreMesh(
        core_axis_name="c", subcore_axis_name="s",
        num_cores=NC, num_subcores=NS),
    compiler_params=pltpu.CompilerParams(
        needs_layout_passes=False, use_tc_tiling_on_sc=False),
    scratch_types=[
        pltpu.VMEM((E_PER_W,), jnp.int32),
        pltpu.VMEM((E_PER_W,), jnp.int32),
        pltpu.VMEM((NGROUPS, GROUP), jnp.int32),
        pltpu.VMEM((E_PER_W, F), jnp.float32),
        pltpu.VMEM((2 * GROUP, F), jnp.float32),
        pltpu.VMEM((NGROUPS, GROUP), jnp.float32),
        pltpu.VMEM((N_NODES,), jnp.float32),
        pltpu.VMEM((GROUP,), jnp.float32),
        pltpu.VMEM((N_ACC // NS,), jnp.float32),
        pltpu.VMEM_SHARED((N_ACC,), jnp.float32),
        pltpu.VMEM_SHARED((N_ACC,), jnp.float32),
        pltpu.SemaphoreType.DMA,
        pltpu.SemaphoreType.DMA,
        pltpu.SemaphoreType.DMA,
    ],
  )(_sc_body)


def kernel(x, edge_index, edge_attr, W_nn, b_nn, root, bias):
    y, c2 = _prep(x, W_nn, b_nn[None, :])
    c = c2.reshape(N_NODES)

    tot, cnt = _get_sc_core()(
        edge_index.astype(jnp.int32), edge_attr, y, c)

    return _final(x, root, bias[None, :],
                  tot.reshape(NC, N_ACC, 1), cnt.reshape(NC, N_ACC, 1))


# speedup vs baseline: 1.2343x; 1.2343x over previous
import functools

import jax
import jax.numpy as jnp
from jax import lax
from jax.experimental import pallas as pl
from jax.experimental.pallas import tpu as pltpu
from jax.experimental.pallas import tpu_sc as plsc

N_NODES = 10000
N_EDGES = 160000
F = 16

NC = 2
NS = 16
NW = NC * NS
GROUP = 128
NGROUPS = 40
E_PER_W = GROUP * NGROUPS
E_REAL_W = N_EDGES // NW
N_ACC = 10240
DUMMY = N_ACC - 1


def _prep_body(x_ref, wnn_ref, bnn_ref, y_ref, c_ref):
    k = lax.broadcasted_iota(jnp.int32, (F * F, F), 0)
    i = lax.broadcasted_iota(jnp.int32, (F * F, F), 1)
    sel = jnp.where(k // F == i, 1.0, 0.0)
    wsum = wnn_ref[...] @ sel
    bsum = bnn_ref[...] @ sel
    xv = x_ref[...]
    y_ref[...] = lax.dot_general(xv, wsum, (((1,), (1,)), ((), ())))
    c_ref[...] = jnp.sum(xv * bsum, axis=1)


_prep = pl.pallas_call(
    _prep_body,
    out_shape=(
        jax.ShapeDtypeStruct((N_NODES, F), jnp.float32),
        jax.ShapeDtypeStruct((N_NODES,), jnp.float32),
    ),
)


def _final_body(x_ref, root_ref, bias_ref, t_ref, c_ref, o_ref):
    xr_t = lax.dot_general(
        root_ref[...], x_ref[...], (((0,), (1,)), ((), ())))
    tot = t_ref[0] + t_ref[1]
    cnt = c_ref[0] + c_ref[1]
    mean = jnp.where(cnt > 0, tot / (cnt * float(F)), 0.0)
    o_ref[...] = xr_t + bias_ref[...] + mean[None, :N_NODES]


_final = pl.pallas_call(
    _final_body,
    out_shape=jax.ShapeDtypeStruct((F, N_NODES), jnp.float32),
    grid=(1,),
    in_specs=[
        pl.BlockSpec((N_NODES, F), lambda i: (0, 0)),
        pl.BlockSpec((F, F), lambda i: (0, 0)),
        pl.BlockSpec((F, 1), lambda i: (0, 0)),
        pl.BlockSpec((NC, N_ACC), lambda i: (0, 0)),
        pl.BlockSpec((NC, N_ACC), lambda i: (0, 0)),
    ],
    out_specs=pl.BlockSpec((F, N_NODES), lambda i: (0, 0)),
)


def _sc_body(ei_hbm, ea_hbm, y_hbm, c_hbm,
             tot_out, cnt_out,
             row_v, col1_v, col_v, ea_v, yr_v, s_v, c_v, ones_v, zv,
             tot_sh, cnt_sh, sem_st, sem_g, sem_s):
    cid = lax.axis_index("c")
    sid = lax.axis_index("s")
    wid = cid * NS + sid

    iota = lax.iota(jnp.int32, 16)
    zeros16 = jnp.zeros((16,), jnp.float32)
    ones16 = jnp.ones((16,), jnp.float32)
    zeros16i = jnp.zeros((16,), jnp.int32)
    dummy16 = jnp.full((16,), DUMMY, jnp.int32)

    ntail = (E_PER_W - E_REAL_W) // 16 + 1
    for k in range(ntail):
        row_v[pl.ds(E_PER_W - 16 * ntail + 16 * k, 16)] = zeros16i
        col1_v[pl.ds(E_PER_W - 16 * ntail + 16 * k, 16)] = dummy16

    cp0 = pltpu.async_copy(
        ei_hbm.at[0].at[pl.ds(wid * E_REAL_W, E_REAL_W)],
        row_v.at[pl.ds(0, E_REAL_W)], sem_st)
    cp1 = pltpu.async_copy(
        ei_hbm.at[1].at[pl.ds(wid * E_REAL_W, E_REAL_W)],
        col1_v.at[pl.ds(0, E_REAL_W)], sem_st)
    cp2 = pltpu.async_copy(c_hbm, c_v, sem_st)
    cp3 = pltpu.async_copy(
        ea_hbm.at[pl.ds(wid * E_REAL_W, E_REAL_W)],
        ea_v.at[pl.ds(0, E_REAL_W)], sem_st)

    for k in range(N_ACC // NS // 16):
        zv[pl.ds(k * 16, 16)] = zeros16
    for k in range(GROUP // 16):
        ones_v[pl.ds(k * 16, 16)] = ones16
    cp0.wait()
    cp1.wait()
    cp2.wait()
    cp3.wait()
    for t in range(E_PER_W // 16):
        col_v[t // 8, pl.ds((t % 8) * 16, 16)] = col1_v[pl.ds(t * 16, 16)]
    pltpu.sync_copy(zv, tot_sh.at[pl.ds(sid * (N_ACC // NS), N_ACC // NS)])
    pltpu.sync_copy(zv, cnt_sh.at[pl.ds(sid * (N_ACC // NS), N_ACC // NS)])
    plsc.subcore_barrier()

    def gather_start(g, base):
        pltpu.async_copy(
            y_hbm.at[row_v.at[pl.ds(g * GROUP, GROUP)]],
            yr_v.at[pl.ds(base, GROUP)], sem_g)

    def gather_wait(g, base):
        pltpu.make_async_copy(
            y_hbm.at[row_v.at[pl.ds(g * GROUP, GROUP)]],
            yr_v.at[pl.ds(base, GROUP)], sem_g).wait()

    def compute_group(g, base):
        for j in range(GROUP // 16):
            el = iota + (j * 16)
            eg = el + g * GROUP
            ey = el + base
            ridx = plsc.load_gather(row_v, [eg])
            acc = plsc.load_gather(c_v, [ridx])
            for i in range(F):
                ii = jnp.full((16,), i, jnp.int32)
                yc = plsc.load_gather(yr_v, [ey, ii])
                ec = plsc.load_gather(ea_v, [eg, ii])
                acc = acc + yc * ec
            s_v[g, pl.ds(j * 16, 16)] = acc
        pltpu.async_copy(s_v.at[g], tot_sh.at[col_v.at[g]], sem_s, add=True)
        pltpu.async_copy(ones_v, cnt_sh.at[col_v.at[g]], sem_s, add=True)

    gather_start(0, 0)

    def pair_body(k, _):
        g0 = k * 2
        g1 = g0 + 1
        gather_wait(g0, 0)
        gather_start(g1, GROUP)
        compute_group(g0, 0)
        gather_wait(g1, GROUP)
        gather_start(jnp.minimum(g0 + 2, NGROUPS - 2), 0)
        compute_group(g1, GROUP)
        return ()

    lax.fori_loop(0, NGROUPS // 2, pair_body, ())
    gather_wait(NGROUPS - 2, 0)

    def drain_body(g, _):
        pltpu.make_async_copy(s_v.at[g], tot_sh.at[col_v.at[g]], sem_s).wait()
        pltpu.make_async_copy(ones_v, cnt_sh.at[col_v.at[g]], sem_s).wait()
        return ()

    lax.fori_loop(0, NGROUPS, drain_body, ())
    plsc.subcore_barrier()

    @pl.when(sid == 0)
    def _():
        pltpu.sync_copy(tot_sh, tot_out.at[cid])
        pltpu.sync_copy(cnt_sh, cnt_out.at[cid])


@functools.cache
def _get_sc_core():
  return functools.partial(
    pl.kernel,
    out_type=(
        jax.ShapeDtypeStruct((NC, N_ACC), jnp.float32),
        jax.ShapeDtypeStruct((NC, N_ACC), jnp.float32),
    ),
    mesh=plsc.VectorSubcoreMesh(
        core_axis_name="c", subcore_axis_name="s",
        num_cores=NC, num_subcores=NS),
    compiler_params=pltpu.CompilerParams(
        needs_layout_passes=False, use_tc_tiling_on_sc=False),
    scratch_types=[
        pltpu.VMEM((E_PER_W,), jnp.int32),
        pltpu.VMEM((E_PER_W,), jnp.int32),
        pltpu.VMEM((NGROUPS, GROUP), jnp.int32),
        pltpu.VMEM((E_PER_W, F), jnp.float32),
        pltpu.VMEM((2 * GROUP, F), jnp.float32),
        pltpu.VMEM((NGROUPS, GROUP), jnp.float32),
        pltpu.VMEM((N_NODES,), jnp.float32),
        pltpu.VMEM((GROUP,), jnp.float32),
        pltpu.VMEM((N_ACC // NS,), jnp.float32),
        pltpu.VMEM_SHARED((N_ACC,), jnp.float32),
        pltpu.VMEM_SHARED((N_ACC,), jnp.float32),
        pltpu.SemaphoreType.DMA,
        pltpu.SemaphoreType.DMA,
        pltpu.SemaphoreType.DMA,
    ],
  )(_sc_body)


def kernel(x, edge_index, edge_attr, W_nn, b_nn, root, bias):
    y, c2 = _prep(x, W_nn, b_nn[None, :])

    tot, cnt = _get_sc_core()(
        edge_index.astype(jnp.int32), edge_attr, y, c2)

    out_t = _final(x, root, bias[:, None], tot, cnt)
    return out_t.T
